# Initial kernel scaffold; baseline (speedup 1.0000x reference)
import jax
import jax.numpy as jnp
from jax.experimental import pallas as pl


def kernel(seq1, seq2, adj, sparse, msk, samp_bias1, samp_bias2, W1, b1, W2, b2, W3, b3, Wd):
    raise NotImplementedError("write your pallas kernel here")


# speedup vs baseline: 1.2594x; 1.2594x over previous
import jax
import jax.numpy as jnp
from jax.experimental import pallas as pl

N_NODES = 10000
N_H = 256


def _prop(x, src, dst, rdeg):
    agg = jnp.zeros_like(x).at[dst].add(x[src])
    return (x + agg) * rdeg


def _fwd(x, src, dst, rdeg, W1, b1, W2, b2, W3e, b3e):
    h = jax.nn.relu(_prop(x @ W1 + b1, src, dst, rdeg))
    h = jax.nn.relu(_prop(h @ W2 + b2, src, dst, rdeg))
    return _prop(h @ W3e + b3e, src, dst, rdeg)


def _tail_kernel(h1_ref, h2_ref, msk_ref, sb1_ref, sb2_ref, wd_ref, out_ref):
    h1 = h1_ref[...]
    h2 = h2_ref[...]
    msk = msk_ref[...]
    c = msk @ h1 / jnp.sum(msk)
    c = jax.nn.sigmoid(c)
    cw = c @ wd_ref[...]
    sc1 = jnp.sum(h1 * cw, axis=1) + sb1_ref[0, :]
    sc2 = jnp.sum(h2 * cw, axis=1) + sb2_ref[0, :]
    out_ref[0, :N_NODES] = sc1
    out_ref[0, N_NODES:] = sc2


def kernel(seq1, seq2, adj, sparse, msk, samp_bias1, samp_bias2, W1, b1, W2, b2, W3, b3, Wd):
    src, dst = adj[0], adj[1]
    deg = jnp.zeros((N_NODES,), jnp.float32).at[dst].add(1.0)
    rdeg = (1.0 / (deg + 1.0))[:, None]
    W3e, b3e = W3[:, ::2], b3[::2]
    h_1 = _fwd(seq1[0], src, dst, rdeg, W1, b1, W2, b2, W3e, b3e)
    h_2 = _fwd(seq2[0], src, dst, rdeg, W1, b1, W2, b2, W3e, b3e)
    out = pl.pallas_call(
        _tail_kernel,
        out_shape=jax.ShapeDtypeStruct((1, 2 * N_NODES), jnp.float32),
    )(h_1, h_2, msk, samp_bias1, samp_bias2, Wd)
    return out


# speedup vs baseline: 3.5451x; 2.8150x over previous
import jax
import jax.numpy as jnp
from jax import lax
from jax.experimental import pallas as pl
from jax.experimental.pallas import tpu as pltpu
from jax.experimental.pallas import tpu_sc as plsc

N = 10000
E = 320000
NIN = 128
NH = 256
HALF = 128
NS = 16
CH = 128
NCHUNK = E // CH
FULL_K = NCHUNK // NS
TAIL = NCHUNK - FULL_K * NS
RB = 1000
FL = 624
FREST = N - NS * FL

_mesh = plsc.VectorSubcoreMesh(core_axis_name="c", subcore_axis_name="s",
                               num_cores=2, num_subcores=NS)


def _deg_body(dst_hbm, deg_out, acc, dstbuf, onesbuf, zbuf):
    cpass = lax.axis_index("c")
    sid = lax.axis_index("s")
    ones16 = jnp.full((16,), 1.0, jnp.float32)
    zeros16 = jnp.zeros((16,), jnp.float32)
    for r in range(CH):
        onesbuf[r, :] = ones16
    for r in range(FL):
        zbuf[r, :] = zeros16
    pltpu.sync_copy(zbuf, acc.at[pl.ds(sid * FL, FL)])
    @pl.when(sid == 0)
    def _():
        pltpu.sync_copy(zbuf.at[pl.ds(0, FREST)], acc.at[pl.ds(NS * FL, FREST)])
    plsc.subcore_barrier()
    half_chunks = NCHUNK // 2
    base = cpass * half_chunks

    def step(k, _):
        c = base + sid + k * NS
        pltpu.sync_copy(dst_hbm.at[pl.ds(c * CH, CH)], dstbuf)
        pltpu.sync_copy(onesbuf, acc.at[dstbuf], add=True)
        return 0

    nfull = half_chunks // NS
    lax.fori_loop(0, nfull, step, 0)
    rem = half_chunks - nfull * NS
    @pl.when(sid < rem)
    def _():
        c = base + nfull * NS + sid
        pltpu.sync_copy(dst_hbm.at[pl.ds(c * CH, CH)], dstbuf)
        pltpu.sync_copy(onesbuf, acc.at[dstbuf], add=True)
    plsc.subcore_barrier()
    pltpu.sync_copy(acc.at[pl.ds(sid * FL, FL)],
                    deg_out.at[cpass, pl.ds(sid * FL, FL)])
    @pl.when(sid == 0)
    def _():
        pltpu.sync_copy(acc.at[pl.ds(NS * FL, FREST)],
                        deg_out.at[cpass, pl.ds(NS * FL, FREST)])


_deg_call = pl.kernel(
    _deg_body,
    out_type=jax.ShapeDtypeStruct((2, N, 16), jnp.float32),
    mesh=_mesh,
    scratch_types=[
        pltpu.VMEM_SHARED((N, 16), jnp.float32),
        pltpu.VMEM((CH,), jnp.int32),
        pltpu.VMEM((CH, 16), jnp.float32),
        pltpu.VMEM((FL, 16), jnp.float32),
    ],
    compiler_params=pltpu.CompilerParams(use_tc_tiling_on_sc=False),
)


def _prop_body(p_lo, p_hi, src_hbm, dst_hbm, s_lo, s_hi, acc,
               srcbuf, dstbuf, idxbuf, rowsbuf, gsem):
    cpass = lax.axis_index("c")
    sid = lax.axis_index("s")
    row0 = cpass * N
    for h, (p_h, s_h) in enumerate(((p_lo, s_lo), (p_hi, s_hi))):
        pltpu.sync_copy(p_h.at[pl.ds(row0 + sid * FL, FL)],
                        acc.at[pl.ds(sid * FL, FL)])
        @pl.when(sid == 0)
        def _():
            pltpu.sync_copy(p_h.at[pl.ds(row0 + NS * FL, FREST)],
                            acc.at[pl.ds(NS * FL, FREST)])
        plsc.subcore_barrier()

        def chunk(c):
            pltpu.sync_copy(src_hbm.at[pl.ds(c * CH, CH)], srcbuf)
            pltpu.sync_copy(dst_hbm.at[pl.ds(c * CH, CH)], dstbuf)
            for i in range(CH // 16):
                s = srcbuf[pl.ds(i * 16, 16)]
                idxbuf[pl.ds(i * 16, 16)] = s + row0
            pltpu.async_copy(p_h.at[idxbuf], rowsbuf, gsem).wait()
            pltpu.sync_copy(rowsbuf, acc.at[dstbuf], add=True)

        def step(k, _):
            chunk(sid + k * NS)
            return 0

        lax.fori_loop(0, FULL_K, step, 0)
        @pl.when(sid < TAIL)
        def _():
            chunk(FULL_K * NS + sid)
        plsc.subcore_barrier()
        pltpu.sync_copy(acc.at[pl.ds(sid * FL, FL)],
                        s_h.at[pl.ds(row0 + sid * FL, FL)])
        @pl.when(sid == 0)
        def _():
            pltpu.sync_copy(acc.at[pl.ds(NS * FL, FREST)],
                            s_h.at[pl.ds(row0 + NS * FL, FREST)])
        plsc.subcore_barrier()


_prop_call = pl.kernel(
    _prop_body,
    out_type=(jax.ShapeDtypeStruct((2 * N, HALF), jnp.float32),
              jax.ShapeDtypeStruct((2 * N, HALF), jnp.float32)),
    mesh=_mesh,
    scratch_types=[
        pltpu.VMEM_SHARED((N, HALF), jnp.float32),
        pltpu.VMEM((CH,), jnp.int32),
        pltpu.VMEM((CH,), jnp.int32),
        pltpu.VMEM((CH,), jnp.int32),
        pltpu.VMEM((CH, HALF), jnp.float32),
        pltpu.SemaphoreType.DMA,
    ],
)


def _mm0_body(x_ref, w_ref, b_ref, lo_ref, hi_ref):
    r = jnp.dot(x_ref[...], w_ref[...],
                preferred_element_type=jnp.float32) + b_ref[...]
    lo_ref[...] = r[:, :HALF]
    hi_ref[...] = r[:, HALF:]


def _mm0(x, w, b):
    return pl.pallas_call(
        _mm0_body,
        grid=(2 * N // RB,),
        in_specs=[
            pl.BlockSpec((RB, NIN), lambda i: (i, 0)),
            pl.BlockSpec((NIN, NH), lambda i: (0, 0)),
            pl.BlockSpec((1, NH), lambda i: (0, 0)),
        ],
        out_specs=[
            pl.BlockSpec((RB, HALF), lambda i: (i, 0)),
            pl.BlockSpec((RB, HALF), lambda i: (i, 0)),
        ],
        out_shape=[
            jax.ShapeDtypeStruct((2 * N, HALF), jnp.float32),
            jax.ShapeDtypeStruct((2 * N, HALF), jnp.float32),
        ],
    )(x, w, b)


def _rdeg_of(da_ref, db_ref):
    return 1.0 / (da_ref[...][:, :1] + db_ref[...][:, :1] + 1.0)


def _fused_body(lo_ref, hi_ref, da_ref, db_ref, w_ref, b_ref, olo_ref, ohi_ref):
    s = jnp.concatenate([lo_ref[...], hi_ref[...]], axis=1)
    h = jax.nn.relu(s * _rdeg_of(da_ref, db_ref))
    r = jnp.dot(h, w_ref[...], preferred_element_type=jnp.float32) + b_ref[...]
    olo_ref[...] = r[:, :HALF]
    ohi_ref[...] = r[:, HALF:]


def _fused_mm(s_lo, s_hi, dega, degb, w, b):
    return pl.pallas_call(
        _fused_body,
        grid=(2 * N // RB,),
        in_specs=[
            pl.BlockSpec((RB, HALF), lambda i: (i, 0)),
            pl.BlockSpec((RB, HALF), lambda i: (i, 0)),
            pl.BlockSpec((RB, 16), lambda i: (lax.rem(i, N // RB), 0)),
            pl.BlockSpec((RB, 16), lambda i: (lax.rem(i, N // RB), 0)),
            pl.BlockSpec((NH, NH), lambda i: (0, 0)),
            pl.BlockSpec((1, NH), lambda i: (0, 0)),
        ],
        out_specs=[
            pl.BlockSpec((RB, HALF), lambda i: (i, 0)),
            pl.BlockSpec((RB, HALF), lambda i: (i, 0)),
        ],
        out_shape=[
            jax.ShapeDtypeStruct((2 * N, HALF), jnp.float32),
            jax.ShapeDtypeStruct((2 * N, HALF), jnp.float32),
        ],
    )(s_lo, s_hi, dega, degb, w, b)


def _colsum_body(lo_ref, hi_ref, da_ref, db_ref, m_ref, cs_ref, ms_ref):
    i = pl.program_id(0)
    @pl.when(i == 0)
    def _():
        cs_ref[...] = jnp.zeros_like(cs_ref)
        ms_ref[...] = jnp.zeros_like(ms_ref)
    s = jnp.concatenate([lo_ref[...], hi_ref[...]], axis=1)
    h = s * _rdeg_of(da_ref, db_ref)
    m = m_ref[0]
    cs_ref[...] += jnp.dot(m, h, preferred_element_type=jnp.float32)
    ms_ref[...] += jnp.sum(m)


def _colsum(s_lo, s_hi, dega, degb, msk):
    return pl.pallas_call(
        _colsum_body,
        grid=(N // RB,),
        in_specs=[
            pl.BlockSpec((RB, HALF), lambda i: (i, 0)),
            pl.BlockSpec((RB, HALF), lambda i: (i, 0)),
            pl.BlockSpec((RB, 16), lambda i: (i, 0)),
            pl.BlockSpec((RB, 16), lambda i: (i, 0)),
            pl.BlockSpec((1, 1, RB), lambda i: (i, 0, 0)),
        ],
        out_specs=[
            pl.BlockSpec((1, NH), lambda i: (0, 0)),
            pl.BlockSpec((1, NH), lambda i: (0, 0)),
        ],
        out_shape=[
            jax.ShapeDtypeStruct((1, NH), jnp.float32),
            jax.ShapeDtypeStruct((1, NH), jnp.float32),
        ],
    )(s_lo, s_hi, dega, degb, msk.reshape(N // RB, 1, RB))


def _scores_body(lo_ref, hi_ref, da_ref, db_ref, cs_ref, ms_ref, wd_ref,
                 sb_ref, o_ref):
    c = jax.nn.sigmoid(cs_ref[...] / ms_ref[...])
    cw = jnp.dot(c, wd_ref[...], preferred_element_type=jnp.float32)
    s = jnp.concatenate([lo_ref[...], hi_ref[...]], axis=1)
    h = s * _rdeg_of(da_ref, db_ref)
    o_ref[...] = jnp.sum(h * cw, axis=1, keepdims=True) + sb_ref[...]


def _scores(s_lo, s_hi, dega, degb, cs, ms, wd, sb):
    return pl.pallas_call(
        _scores_body,
        grid=(2 * N // RB,),
        in_specs=[
            pl.BlockSpec((RB, HALF), lambda i: (i, 0)),
            pl.BlockSpec((RB, HALF), lambda i: (i, 0)),
            pl.BlockSpec((RB, 16), lambda i: (lax.rem(i, N // RB), 0)),
            pl.BlockSpec((RB, 16), lambda i: (lax.rem(i, N // RB), 0)),
            pl.BlockSpec((1, NH), lambda i: (0, 0)),
            pl.BlockSpec((1, NH), lambda i: (0, 0)),
            pl.BlockSpec((NH, NH), lambda i: (0, 0)),
            pl.BlockSpec((RB, 1), lambda i: (i, 0)),
        ],
        out_specs=pl.BlockSpec((RB, 1), lambda i: (i, 0)),
        out_shape=jax.ShapeDtypeStruct((2 * N, 1), jnp.float32),
    )(s_lo, s_hi, dega, degb, cs, ms, wd, sb)


def kernel(seq1, seq2, adj, sparse, msk, samp_bias1, samp_bias2,
           W1, b1, W2, b2, W3, b3, Wd):
    src, dst = adj[0], adj[1]
    W3e, b3e = W3[:, ::2], b3[::2]
    x2 = jnp.concatenate([seq1, seq2], axis=1)[0]
    sb = jnp.concatenate([samp_bias1, samp_bias2], axis=1).reshape(2 * N, 1)

    deg = _deg_call(dst)
    dega, degb = deg[0], deg[1]

    p0_lo, p0_hi = _mm0(x2, W1, b1.reshape(1, NH))
    s0_lo, s0_hi = _prop_call(p0_lo, p0_hi, src, dst)
    p1_lo, p1_hi = _fused_mm(s0_lo, s0_hi, dega, degb, W2, b2.reshape(1, NH))
    s1_lo, s1_hi = _prop_call(p1_lo, p1_hi, src, dst)
    p2_lo, p2_hi = _fused_mm(s1_lo, s1_hi, dega, degb, W3e, b3e.reshape(1, NH))
    s2_lo, s2_hi = _prop_call(p2_lo, p2_hi, src, dst)

    cs, ms = _colsum(s2_lo[:N], s2_hi[:N], dega, degb, msk)
    out = _scores(s2_lo, s2_hi, dega, degb, cs, ms, Wd, sb)
    return out.reshape(1, 2 * N)


# speedup vs baseline: 5.1687x; 1.4580x over previous
import jax
import jax.numpy as jnp
from jax import lax
from jax.experimental import pallas as pl
from jax.experimental.pallas import tpu as pltpu
from jax.experimental.pallas import tpu_sc as plsc

N = 10000
E = 320000
NIN = 128
NH = 256
HALF = 128
NS = 16
CH = 128
NCHUNK = E // CH
FULL_K = NCHUNK // NS
TAIL = NCHUNK - FULL_K * NS
RB = 1000
FL = 624
FREST = N - NS * FL

_mesh = plsc.VectorSubcoreMesh(core_axis_name="c", subcore_axis_name="s",
                               num_cores=2, num_subcores=NS)


def _deg_body(dst_hbm, deg_out, acc, dstbuf, onesbuf, zbuf):
    cpass = lax.axis_index("c")
    sid = lax.axis_index("s")
    ones16 = jnp.full((16,), 1.0, jnp.float32)
    zeros16 = jnp.zeros((16,), jnp.float32)
    for r in range(CH):
        onesbuf[r, :] = ones16
    for r in range(FL):
        zbuf[r, :] = zeros16
    pltpu.sync_copy(zbuf, acc.at[pl.ds(sid * FL, FL)])
    @pl.when(sid == 0)
    def _():
        pltpu.sync_copy(zbuf.at[pl.ds(0, FREST)], acc.at[pl.ds(NS * FL, FREST)])
    plsc.subcore_barrier()
    half_chunks = NCHUNK // 2
    base = cpass * half_chunks

    def step(k, _):
        c = base + sid + k * NS
        pltpu.sync_copy(dst_hbm.at[pl.ds(c * CH, CH)], dstbuf)
        pltpu.sync_copy(onesbuf, acc.at[dstbuf], add=True)
        return 0

    nfull = half_chunks // NS
    lax.fori_loop(0, nfull, step, 0)
    rem = half_chunks - nfull * NS
    @pl.when(sid < rem)
    def _():
        c = base + nfull * NS + sid
        pltpu.sync_copy(dst_hbm.at[pl.ds(c * CH, CH)], dstbuf)
        pltpu.sync_copy(onesbuf, acc.at[dstbuf], add=True)
    plsc.subcore_barrier()
    pltpu.sync_copy(acc.at[pl.ds(sid * FL, FL)],
                    deg_out.at[cpass, pl.ds(sid * FL, FL)])
    @pl.when(sid == 0)
    def _():
        pltpu.sync_copy(acc.at[pl.ds(NS * FL, FREST)],
                        deg_out.at[cpass, pl.ds(NS * FL, FREST)])


_deg_call = pl.kernel(
    _deg_body,
    out_type=jax.ShapeDtypeStruct((2, N, 16), jnp.float32),
    mesh=_mesh,
    scratch_types=[
        pltpu.VMEM_SHARED((N, 16), jnp.float32),
        pltpu.VMEM((CH,), jnp.int32),
        pltpu.VMEM((CH, 16), jnp.float32),
        pltpu.VMEM((FL, 16), jnp.float32),
    ],
    compiler_params=pltpu.CompilerParams(use_tc_tiling_on_sc=False),
)


NCHT = 157
EPT_PAD = NCHT * CH
E_PAD = NS * EPT_PAD


SEGS = ((0, 53), (53, 53), (106, 51))
SEGMAX = 53


def _prop_body(p_lo, p_hi, src_hbm, dst_hbm, s_lo, s_hi, acc,
               srcall, dstall, rows0, rows1, gs0, gs1):
    cpass = lax.axis_index("c")
    sid = lax.axis_index("s")
    row0 = cpass * N

    for p_h, s_h in ((p_lo, s_lo), (p_hi, s_hi)):
        pltpu.sync_copy(p_h.at[pl.ds(row0 + sid * FL, FL)],
                        acc.at[pl.ds(sid * FL, FL)])
        @pl.when(sid == 0)
        def _():
            pltpu.sync_copy(p_h.at[pl.ds(row0 + NS * FL, FREST)],
                            acc.at[pl.ds(NS * FL, FREST)])
        plsc.subcore_barrier()

        for sbase, slen in SEGS:
            pltpu.sync_copy(src_hbm.at[pl.ds(sid * NCHT + sbase, slen)],
                            srcall.at[pl.ds(0, slen)])
            pltpu.sync_copy(dst_hbm.at[pl.ds(sid * NCHT + sbase, slen)],
                            dstall.at[pl.ds(0, slen)])

            def xform(j, _):
                for i in range(CH // 16):
                    srcall[j, pl.ds(i * 16, 16)] = (
                        srcall[j, pl.ds(i * 16, 16)] + row0)
                return 0

            lax.fori_loop(0, slen, xform, 0)

            pltpu.async_copy(p_h.at[srcall.at[0]], rows0, gs0)

            def pair(k, _):
                j = 2 * k
                pltpu.make_async_copy(p_h.at[srcall.at[j]], rows0, gs0).wait()
                pltpu.async_copy(p_h.at[srcall.at[j + 1]], rows1, gs1)
                pltpu.sync_copy(rows0, acc.at[dstall.at[j]], add=True)
                pltpu.make_async_copy(
                    p_h.at[srcall.at[j + 1]], rows1, gs1).wait()
                pltpu.async_copy(p_h.at[srcall.at[j + 2]], rows0, gs0)
                pltpu.sync_copy(rows1, acc.at[dstall.at[j + 1]], add=True)
                return 0

            lax.fori_loop(0, (slen - 1) // 2, pair, 0)
            pltpu.make_async_copy(
                p_h.at[srcall.at[slen - 1]], rows0, gs0).wait()
            pltpu.sync_copy(rows0, acc.at[dstall.at[slen - 1]], add=True)
        plsc.subcore_barrier()
        pltpu.sync_copy(acc.at[pl.ds(sid * FL, FL)],
                        s_h.at[pl.ds(row0 + sid * FL, FL)])
        @pl.when(sid == 0)
        def _():
            pltpu.sync_copy(acc.at[pl.ds(NS * FL, FREST)],
                            s_h.at[pl.ds(row0 + NS * FL, FREST)])
        plsc.subcore_barrier()


_prop_call = pl.kernel(
    _prop_body,
    out_type=(jax.ShapeDtypeStruct((2 * N, HALF), jnp.float32),
              jax.ShapeDtypeStruct((2 * N, HALF), jnp.float32)),
    mesh=_mesh,
    scratch_types=[
        pltpu.VMEM_SHARED((N + 16, HALF), jnp.float32),
        pltpu.VMEM((SEGMAX, CH), jnp.int32),
        pltpu.VMEM((SEGMAX, CH), jnp.int32),
        pltpu.VMEM((CH, HALF), jnp.float32),
        pltpu.VMEM((CH, HALF), jnp.float32),
        pltpu.SemaphoreType.DMA,
        pltpu.SemaphoreType.DMA,
    ],
    compiler_params=pltpu.CompilerParams(use_tc_tiling_on_sc=False),
)


def _mm0_body(x_ref, w_ref, b_ref, lo_ref, hi_ref):
    r = jnp.dot(x_ref[...], w_ref[...],
                preferred_element_type=jnp.float32) + b_ref[...]
    lo_ref[...] = r[:, :HALF]
    hi_ref[...] = r[:, HALF:]


def _mm0(x, w, b):
    return pl.pallas_call(
        _mm0_body,
        grid=(2 * N // RB,),
        in_specs=[
            pl.BlockSpec((RB, NIN), lambda i: (i, 0)),
            pl.BlockSpec((NIN, NH), lambda i: (0, 0)),
            pl.BlockSpec((1, NH), lambda i: (0, 0)),
        ],
        out_specs=[
            pl.BlockSpec((RB, HALF), lambda i: (i, 0)),
            pl.BlockSpec((RB, HALF), lambda i: (i, 0)),
        ],
        out_shape=[
            jax.ShapeDtypeStruct((2 * N, HALF), jnp.float32),
            jax.ShapeDtypeStruct((2 * N, HALF), jnp.float32),
        ],
    )(x, w, b)


def _rdeg_of(da_ref, db_ref):
    return 1.0 / (da_ref[...][:, :1] + db_ref[...][:, :1] + 1.0)


def _fused_body(lo_ref, hi_ref, da_ref, db_ref, w_ref, b_ref, olo_ref, ohi_ref):
    s = jnp.concatenate([lo_ref[...], hi_ref[...]], axis=1)
    h = jax.nn.relu(s * _rdeg_of(da_ref, db_ref))
    r = jnp.dot(h, w_ref[...], preferred_element_type=jnp.float32) + b_ref[...]
    olo_ref[...] = r[:, :HALF]
    ohi_ref[...] = r[:, HALF:]


def _fused_mm(s_lo, s_hi, dega, degb, w, b):
    return pl.pallas_call(
        _fused_body,
        grid=(2 * N // RB,),
        in_specs=[
            pl.BlockSpec((RB, HALF), lambda i: (i, 0)),
            pl.BlockSpec((RB, HALF), lambda i: (i, 0)),
            pl.BlockSpec((RB, 16), lambda i: (lax.rem(i, N // RB), 0)),
            pl.BlockSpec((RB, 16), lambda i: (lax.rem(i, N // RB), 0)),
            pl.BlockSpec((NH, NH), lambda i: (0, 0)),
            pl.BlockSpec((1, NH), lambda i: (0, 0)),
        ],
        out_specs=[
            pl.BlockSpec((RB, HALF), lambda i: (i, 0)),
            pl.BlockSpec((RB, HALF), lambda i: (i, 0)),
        ],
        out_shape=[
            jax.ShapeDtypeStruct((2 * N, HALF), jnp.float32),
            jax.ShapeDtypeStruct((2 * N, HALF), jnp.float32),
        ],
    )(s_lo, s_hi, dega, degb, w, b)


def _colsum_body(lo_ref, hi_ref, da_ref, db_ref, m_ref, cs_ref, ms_ref):
    i = pl.program_id(0)
    @pl.when(i == 0)
    def _():
        cs_ref[...] = jnp.zeros_like(cs_ref)
        ms_ref[...] = jnp.zeros_like(ms_ref)
    s = jnp.concatenate([lo_ref[...], hi_ref[...]], axis=1)
    h = s * _rdeg_of(da_ref, db_ref)
    m = m_ref[0]
    cs_ref[...] += jnp.dot(m, h, preferred_element_type=jnp.float32)
    ms_ref[...] += jnp.sum(m)


def _colsum(s_lo, s_hi, dega, degb, msk):
    return pl.pallas_call(
        _colsum_body,
        grid=(N // RB,),
        in_specs=[
            pl.BlockSpec((RB, HALF), lambda i: (i, 0)),
            pl.BlockSpec((RB, HALF), lambda i: (i, 0)),
            pl.BlockSpec((RB, 16), lambda i: (i, 0)),
            pl.BlockSpec((RB, 16), lambda i: (i, 0)),
            pl.BlockSpec((1, 1, RB), lambda i: (i, 0, 0)),
        ],
        out_specs=[
            pl.BlockSpec((1, NH), lambda i: (0, 0)),
            pl.BlockSpec((1, NH), lambda i: (0, 0)),
        ],
        out_shape=[
            jax.ShapeDtypeStruct((1, NH), jnp.float32),
            jax.ShapeDtypeStruct((1, NH), jnp.float32),
        ],
    )(s_lo, s_hi, dega, degb, msk.reshape(N // RB, 1, RB))


def _scores_body(lo_ref, hi_ref, da_ref, db_ref, cs_ref, ms_ref, wd_ref,
                 sb_ref, o_ref):
    c = jax.nn.sigmoid(cs_ref[...] / ms_ref[...])
    cw = jnp.dot(c, wd_ref[...], preferred_element_type=jnp.float32)
    s = jnp.concatenate([lo_ref[...], hi_ref[...]], axis=1)
    h = s * _rdeg_of(da_ref, db_ref)
    o_ref[...] = jnp.sum(h * cw, axis=1, keepdims=True) + sb_ref[...]


def _scores(s_lo, s_hi, dega, degb, cs, ms, wd, sb):
    return pl.pallas_call(
        _scores_body,
        grid=(2 * N // RB,),
        in_specs=[
            pl.BlockSpec((RB, HALF), lambda i: (i, 0)),
            pl.BlockSpec((RB, HALF), lambda i: (i, 0)),
            pl.BlockSpec((RB, 16), lambda i: (lax.rem(i, N // RB), 0)),
            pl.BlockSpec((RB, 16), lambda i: (lax.rem(i, N // RB), 0)),
            pl.BlockSpec((1, NH), lambda i: (0, 0)),
            pl.BlockSpec((1, NH), lambda i: (0, 0)),
            pl.BlockSpec((NH, NH), lambda i: (0, 0)),
            pl.BlockSpec((RB, 1), lambda i: (i, 0)),
        ],
        out_specs=pl.BlockSpec((RB, 1), lambda i: (i, 0)),
        out_shape=jax.ShapeDtypeStruct((2 * N, 1), jnp.float32),
    )(s_lo, s_hi, dega, degb, cs, ms, wd, sb)


def kernel(seq1, seq2, adj, sparse, msk, samp_bias1, samp_bias2,
           W1, b1, W2, b2, W3, b3, Wd):
    src, dst = adj[0], adj[1]
    npad = E_PAD - E
    srcp = jnp.concatenate([src, jnp.zeros((npad,), jnp.int32)]).reshape(-1, CH)
    dstp = jnp.concatenate([dst, jnp.full((npad,), N, jnp.int32)]).reshape(-1, CH)
    W3e, b3e = W3[:, ::2], b3[::2]
    x2 = jnp.concatenate([seq1, seq2], axis=1)[0]
    sb = jnp.concatenate([samp_bias1, samp_bias2], axis=1).reshape(2 * N, 1)

    deg = _deg_call(dst)
    dega, degb = deg[0], deg[1]

    p0_lo, p0_hi = _mm0(x2, W1, b1.reshape(1, NH))
    s0_lo, s0_hi = _prop_call(p0_lo, p0_hi, srcp, dstp)
    p1_lo, p1_hi = _fused_mm(s0_lo, s0_hi, dega, degb, W2, b2.reshape(1, NH))
    s1_lo, s1_hi = _prop_call(p1_lo, p1_hi, srcp, dstp)
    p2_lo, p2_hi = _fused_mm(s1_lo, s1_hi, dega, degb, W3e, b3e.reshape(1, NH))
    s2_lo, s2_hi = _prop_call(p2_lo, p2_hi, srcp, dstp)

    cs, ms = _colsum(s2_lo[:N], s2_hi[:N], dega, degb, msk)
    out = _scores(s2_lo, s2_hi, dega, degb, cs, ms, Wd, sb)
    return out.reshape(1, 2 * N)
